# baseline (speedup 1.0000x reference)
import functools

import jax
import jax.numpy as jnp
from jax import lax
from jax.experimental import pallas as pl
from jax.experimental.pallas import tpu as pltpu
from jax.experimental.pallas import tpu_sc as plsc

BATCH = 16384
D0 = 32
D1 = 32
D2 = 64
DTOT = D0 + D1 + D2
HIDDEN = 128
V = 100000

_NC = 2
_NS = 16
_NW = _NC * _NS
_RPW = DTOT // _NW
_OCHUNK = 4096
_L = 16


def _gather_body(segT, t0T, t1T, t2T, eT,
                 idx_v, row_v, out_r, semI, semR, semO):
    wid = lax.axis_index("s") * _NC + lax.axis_index("c")

    @pl.when((wid & 3) > 0)
    def _():
        t = lax.fori_loop(0, 750 * (wid & 3), lambda i, a: a + 1, 0)
        out_r[pl.ds(0, _L)] = jnp.full((_L,), t, jnp.float32)

    def do_table(tbl, ev_base, ti, base):
        dI = pltpu.async_copy(segT.at[pl.ds(ti, 1)], idx_v, semI)
        pltpu.async_copy(tbl.at[base], row_v, semR)
        dI.wait()

        def row_body(k, carry):
            c = base + k
            pltpu.make_async_copy(tbl.at[base], row_v, semR).wait()

            def chunk_body(h, carry2):
                so = (h % 2) * _OCHUNK

                @pl.when((k > 0) | (h >= 2))
                def _():
                    pltpu.make_async_copy(
                        out_r.at[pl.ds(0, _OCHUNK)],
                        eT.at[ev_base, pl.ds(0, _OCHUNK)], semO).wait()

                def gbody(j, carry):
                    off = j * (_L * 8)
                    ivs = [idx_v[0, pl.ds(h * _OCHUNK + off + t * _L, _L)]
                           for t in range(8)]
                    gs = [plsc.load_gather(row_v, [iv]) for iv in ivs]
                    for t in range(8):
                        out_r[pl.ds(so + off + t * _L, _L)] = gs[t]
                    return carry
                lax.fori_loop(0, _OCHUNK // (_L * 8), gbody, 0, unroll=2)
                pltpu.async_copy(
                    out_r.at[pl.ds(so, _OCHUNK)],
                    eT.at[ev_base + c, pl.ds(h * _OCHUNK, _OCHUNK)], semO)
                return carry2

            lax.fori_loop(0, BATCH // _OCHUNK, chunk_body, 0, unroll=1)

            @pl.when(k < _RPW - 1)
            def _():
                pltpu.async_copy(tbl.at[c + 1], row_v, semR)
            return carry

        lax.fori_loop(0, _RPW, row_body, 0, unroll=1)
        for _ in range(2):
            pltpu.make_async_copy(
                out_r.at[pl.ds(0, _OCHUNK)],
                eT.at[ev_base, pl.ds(0, _OCHUNK)], semO).wait()

    @pl.when(wid < 8)
    def _():
        do_table(t0T, 0, 0, wid * _RPW)

    @pl.when((wid >= 8) & (wid < 16))
    def _():
        do_table(t1T, D0, 1, (wid - 8) * _RPW)

    @pl.when(wid >= 16)
    def _():
        do_table(t2T, D0 + D1, 2, (wid - 16) * _RPW)


@functools.cache
def _make_gather():
    return pl.kernel(
        _gather_body,
        out_type=jax.ShapeDtypeStruct((DTOT, BATCH), jnp.float32),
        mesh=plsc.VectorSubcoreMesh(core_axis_name="c", subcore_axis_name="s"),
        scratch_types=[
            pltpu.VMEM((1, BATCH), jnp.int32),
            pltpu.VMEM((V,), jnp.float32),
            pltpu.VMEM((2 * _OCHUNK,), jnp.float32),
            pltpu.SemaphoreType.DMA,
            pltpu.SemaphoreType.DMA,
            pltpu.SemaphoreType.DMA,
        ],
        compiler_params=pltpu.CompilerParams(needs_layout_passes=False),
    )


_MM_COLS = 4096


def _mm_body(e_ref, w_ref, b_ref, o_ref):
    dn = (((0,), (0,)), ((), ()))
    acc = lax.dot_general(e_ref[...], w_ref[...], dn,
                          preferred_element_type=jnp.float32)
    o_ref[...] = acc + b_ref[...]


_matmul = pl.pallas_call(
    _mm_body,
    grid=(BATCH // _MM_COLS,),
    in_specs=[
        pl.BlockSpec((DTOT, _MM_COLS), lambda i: (0, i)),
        pl.BlockSpec((DTOT, HIDDEN), lambda i: (0, 0)),
        pl.BlockSpec((1, HIDDEN), lambda i: (0, 0)),
    ],
    out_specs=pl.BlockSpec((_MM_COLS, HIDDEN), lambda i: (i, 0)),
    out_shape=jax.ShapeDtypeStruct((BATCH, HIDDEN), jnp.float32),
)


@jax.jit
def kernel(segment_features, lane_table, type_table, length_table, W, b):
    eT = _make_gather()(
        segment_features.astype(jnp.int32).T,
        lane_table.T, type_table.T, length_table.T)
    return _matmul(eT, W.T, b.reshape(1, HIDDEN))


# speedup vs baseline: 1.0024x; 1.0024x over previous
import functools

import jax
import jax.numpy as jnp
from jax import lax
from jax.experimental import pallas as pl
from jax.experimental.pallas import tpu as pltpu
from jax.experimental.pallas import tpu_sc as plsc

BATCH = 16384
D0 = 32
D1 = 32
D2 = 64
DTOT = D0 + D1 + D2
HIDDEN = 128
V = 100000

_NC = 2
_NS = 16
_NW = _NC * _NS
_RPW = DTOT // _NW
_OCHUNK = 4096
_L = 16


def _gather_body(segT, t0T, t1T, t2T, eT,
                 idx_v, row_v, out_r, semI, semR, semO):
    wid = lax.axis_index("s") * _NC + lax.axis_index("c")

    def do_table(tbl, ev_base, ti, base):
        dI = pltpu.async_copy(segT.at[pl.ds(ti, 1)], idx_v, semI)
        pltpu.async_copy(tbl.at[base], row_v, semR)
        dI.wait()

        def row_body(k, carry):
            c = base + k
            pltpu.make_async_copy(tbl.at[base], row_v, semR).wait()

            def chunk_body(h, carry2):
                so = (h % 2) * _OCHUNK

                @pl.when((k > 0) | (h >= 2))
                def _():
                    pltpu.make_async_copy(
                        out_r.at[pl.ds(0, _OCHUNK)],
                        eT.at[ev_base, pl.ds(0, _OCHUNK)], semO).wait()

                def gbody(j, carry):
                    off = j * (_L * 8)
                    ivs = [idx_v[0, pl.ds(h * _OCHUNK + off + t * _L, _L)]
                           for t in range(8)]
                    gs = [plsc.load_gather(row_v, [iv]) for iv in ivs]
                    for t in range(8):
                        out_r[pl.ds(so + off + t * _L, _L)] = gs[t]
                    return carry
                lax.fori_loop(0, _OCHUNK // (_L * 8), gbody, 0, unroll=1)
                pltpu.async_copy(
                    out_r.at[pl.ds(so, _OCHUNK)],
                    eT.at[ev_base + c, pl.ds(h * _OCHUNK, _OCHUNK)], semO)
                return carry2

            lax.fori_loop(0, BATCH // _OCHUNK, chunk_body, 0, unroll=1)

            @pl.when(k < _RPW - 1)
            def _():
                pltpu.async_copy(tbl.at[c + 1], row_v, semR)
            return carry

        lax.fori_loop(0, _RPW, row_body, 0, unroll=1)
        for _ in range(2):
            pltpu.make_async_copy(
                out_r.at[pl.ds(0, _OCHUNK)],
                eT.at[ev_base, pl.ds(0, _OCHUNK)], semO).wait()

    @pl.when(wid < 8)
    def _():
        do_table(t0T, 0, 0, wid * _RPW)

    @pl.when((wid >= 8) & (wid < 16))
    def _():
        do_table(t1T, D0, 1, (wid - 8) * _RPW)

    @pl.when(wid >= 16)
    def _():
        do_table(t2T, D0 + D1, 2, (wid - 16) * _RPW)


@functools.cache
def _make_gather():
    return pl.kernel(
        _gather_body,
        out_type=jax.ShapeDtypeStruct((DTOT, BATCH), jnp.float32),
        mesh=plsc.VectorSubcoreMesh(core_axis_name="c", subcore_axis_name="s"),
        scratch_types=[
            pltpu.VMEM((1, BATCH), jnp.int32),
            pltpu.VMEM((V,), jnp.float32),
            pltpu.VMEM((2 * _OCHUNK,), jnp.float32),
            pltpu.SemaphoreType.DMA,
            pltpu.SemaphoreType.DMA,
            pltpu.SemaphoreType.DMA,
        ],
        compiler_params=pltpu.CompilerParams(needs_layout_passes=False),
    )


_MM_COLS = 4096


def _mm_body(e_ref, w_ref, b_ref, o_ref):
    dn = (((0,), (0,)), ((), ()))
    acc = lax.dot_general(e_ref[...], w_ref[...], dn,
                          preferred_element_type=jnp.float32)
    o_ref[...] = acc + b_ref[...]


_matmul = pl.pallas_call(
    _mm_body,
    grid=(BATCH // _MM_COLS,),
    in_specs=[
        pl.BlockSpec((DTOT, _MM_COLS), lambda i: (0, i)),
        pl.BlockSpec((DTOT, HIDDEN), lambda i: (0, 0)),
        pl.BlockSpec((1, HIDDEN), lambda i: (0, 0)),
    ],
    out_specs=pl.BlockSpec((_MM_COLS, HIDDEN), lambda i: (i, 0)),
    out_shape=jax.ShapeDtypeStruct((BATCH, HIDDEN), jnp.float32),
)


@jax.jit
def kernel(segment_features, lane_table, type_table, length_table, W, b):
    eT = _make_gather()(
        segment_features.astype(jnp.int32).T,
        lane_table.T, type_table.T, length_table.T)
    return _matmul(eT, W.T, b.reshape(1, HIDDEN))
